# Initial kernel scaffold; baseline (speedup 1.0000x reference)
import jax
import jax.numpy as jnp
from jax.experimental import pallas as pl


def kernel(x, table):
    raise NotImplementedError("write your pallas kernel here")


# speedup vs baseline: 1.0955x; 1.0955x over previous
import jax
import jax.numpy as jnp
from jax.experimental import pallas as pl

_BLK = 8


def _add_body(x_ref, t_ref, o_ref):
    o_ref[...] = x_ref[...] + t_ref[...]


def kernel(x, table):
    B, N, D = x.shape
    nd = N * D
    x2 = x.reshape(B, nd)
    out = pl.pallas_call(
        _add_body,
        grid=(B // _BLK,),
        in_specs=[
            pl.BlockSpec((_BLK, nd), lambda i: (i, 0)),
            pl.BlockSpec((_BLK, nd), lambda i: (i, 0)),
        ],
        out_specs=pl.BlockSpec((_BLK, nd), lambda i: (i, 0)),
        out_shape=jax.ShapeDtypeStruct((B, nd), jnp.float32),
    )(x2, table)
    return out.reshape(B, N, D)
